# baseline (speedup 1.0000x reference)
import functools

import jax
import jax.numpy as jnp
from jax import lax
from jax.experimental import pallas as pl
from jax.experimental.pallas import tpu as pltpu
from jax.experimental.pallas import tpu_sc as plsc

B = 4096
N = 200
K = 50
D = 64

NC = 2
NS = 16
NW = NC * NS
L = 16

ROWS = B * K
R_PER_W = ROWS // NW
CH = 128
NCH = R_PER_W // CH


def _sc_gather(table, idx):
    mesh = plsc.VectorSubcoreMesh(core_axis_name="c", subcore_axis_name="s")

    @functools.partial(
        pl.kernel,
        mesh=mesh,
        out_type=jax.ShapeDtypeStruct((ROWS, D), jnp.float32),
        scratch_types=[
            pltpu.VMEM((CH,), jnp.int32),
            pltpu.VMEM((CH,), jnp.int32),
            pltpu.VMEM((CH, D), jnp.float32),
            pltpu.SemaphoreType.DMA,
        ],
        compiler_params=pltpu.CompilerParams(use_tc_tiling_on_sc=False),
    )
    def k(table_hbm, idx_hbm, out_hbm, raw_v, gidx_v, rows_v, sem):
        wid = lax.axis_index("s") * NC + lax.axis_index("c")
        wbase = wid * R_PER_W

        def chunk_body(ci, carry):
            base = wbase + ci * CH
            pltpu.sync_copy(idx_hbm.at[pl.ds(base, CH)], raw_v)

            def vec_body(v, c2):
                pos = base + v * L + lax.iota(jnp.int32, L)
                bidx = lax.div(pos, jnp.int32(K))
                gidx_v[pl.ds(v * L, L)] = (
                    raw_v[pl.ds(v * L, L)] + bidx * jnp.int32(N)
                )
                return c2

            lax.fori_loop(0, CH // L, vec_body, 0)
            pltpu.async_copy(table_hbm.at[gidx_v], rows_v, sem).wait()
            pltpu.sync_copy(rows_v, out_hbm.at[pl.ds(base, CH)])
            return carry

        lax.fori_loop(0, NCH, chunk_body, 0)

    return k(table, idx)


def kernel(reference, indices):
    table = reference.reshape(B * N, D)
    idx = indices.astype(jnp.int32).reshape(ROWS)
    out = _sc_gather(table, idx)
    return out.reshape(B, K, D)


# speedup vs baseline: 1.0918x; 1.0918x over previous
import functools

import jax
import jax.numpy as jnp
from jax import lax
from jax.experimental import pallas as pl
from jax.experimental.pallas import tpu as pltpu
from jax.experimental.pallas import tpu_sc as plsc

B = 4096
N = 200
K = 50
D = 64

NC = 2
NS = 16
NW = NC * NS
L = 16

ROWS = B * K
R_PER_W = ROWS // NW
CH = 800
NCH = R_PER_W // CH
NBUF = 2


def _sc_gather(table, idx):
    mesh = plsc.VectorSubcoreMesh(core_axis_name="c", subcore_axis_name="s")

    @functools.partial(
        pl.kernel,
        mesh=mesh,
        out_type=jax.ShapeDtypeStruct((ROWS, D), jnp.float32),
        scratch_types=[
            pltpu.VMEM((R_PER_W,), jnp.int32),
            pltpu.VMEM((R_PER_W,), jnp.int32),
            [pltpu.VMEM((CH, D), jnp.float32)] * NBUF,
            [pltpu.SemaphoreType.DMA] * NBUF,
            [pltpu.SemaphoreType.DMA] * NBUF,
        ],
        compiler_params=pltpu.CompilerParams(use_tc_tiling_on_sc=False),
    )
    def k(table_hbm, idx_hbm, out_hbm, raw_v, gidx_v, rows, gsem, ssem):
        wid = lax.axis_index("s") * NC + lax.axis_index("c")
        wbase = wid * R_PER_W

        pltpu.sync_copy(idx_hbm.at[pl.ds(wbase, R_PER_W)], raw_v)

        def compute_gidx(c):
            def vec_body(v, c2):
                pos = wbase + c * CH + v * L + lax.iota(jnp.int32, L)
                bidx = lax.div(pos, jnp.int32(K))
                gidx_v[pl.ds(c * CH + v * L, L)] = (
                    raw_v[pl.ds(c * CH + v * L, L)] + bidx * jnp.int32(N)
                )
                return c2

            lax.fori_loop(0, CH // L, vec_body, 0)

        def fire_gather(c, b):
            compute_gidx(c)
            return pltpu.async_copy(
                table_hbm.at[gidx_v.at[pl.ds(c * CH, CH)]], rows[b], gsem[b]
            )

        def fire_store(c, b):
            return pltpu.async_copy(
                rows[b], out_hbm.at[pl.ds(wbase + c * CH, CH)], ssem[b]
            )

        gathers = [None] * NBUF
        stores = [None] * NBUF
        for b in range(NBUF):
            gathers[b] = fire_gather(b, b)
        for c in range(NCH):
            b = c % NBUF
            gathers[b].wait()
            stores[b] = fire_store(c, b)
            nxt = c + NBUF
            if nxt < NCH:
                stores[b].wait()
                gathers[b] = fire_gather(nxt, b)
        for b in range(NBUF):
            stores[(NCH - NBUF + b) % NBUF].wait()

    return k(table, idx)


def kernel(reference, indices):
    table = reference.reshape(B * N, D)
    idx = indices.astype(jnp.int32).reshape(ROWS)
    out = _sc_gather(table, idx)
    return out.reshape(B, K, D)
